# Initial kernel scaffold; baseline (speedup 1.0000x reference)
import jax
import jax.numpy as jnp
from jax.experimental import pallas as pl


def kernel(node_tokens, node_batch, active_nodes, question_tokens, step_counts, step_emb_weight, ln_gamma, ln_beta):
    raise NotImplementedError("write your pallas kernel here")


# speedup vs baseline: 4.7950x; 4.7950x over previous
import functools

import jax
import jax.numpy as jnp
from jax import lax
from jax.experimental import pallas as pl
from jax.experimental.pallas import tpu as pltpu
from jax.experimental.pallas import tpu_sc as plsc

NC = 2
NS = 16
NW = NC * NS
CH = 128
LANES = 16


def _seg_body(num_chunks, Bn, H,
              nt_hbm, nb_hbm, act_hbm, zsum_hbm, zcnt_hbm, ones_hbm,
              out_sum_hbm, out_cnt_hbm,
              data_v, nb_v, act_v, idx_v, ones_v,
              acc_sum, acc_cnt):
    cid = lax.axis_index("c")
    sid = lax.axis_index("s")
    wid = sid * NC + cid

    @pl.when(sid == 0)
    def _():
        pltpu.sync_copy(zsum_hbm, acc_sum)
        pltpu.sync_copy(zcnt_hbm, acc_cnt)

    pltpu.sync_copy(ones_hbm, ones_v)
    plsc.subcore_barrier()

    n_full = num_chunks // NW
    rem = num_chunks % NW
    my_n = n_full + jnp.where(wid < rem, 1, 0)

    def body(j, carry):
        c = wid + NW * j
        base = c * CH
        pltpu.sync_copy(nt_hbm.at[pl.ds(base, CH)], data_v)
        pltpu.sync_copy(nb_hbm.at[pl.ds(base, CH)], nb_v)
        pltpu.sync_copy(act_hbm.at[pl.ds(base, CH)], act_v)
        for i in range(CH // LANES):
            sl = pl.ds(i * LANES, LANES)
            nb16 = nb_v[sl]
            a16 = act_v[sl]
            idx_v[0, sl] = jnp.where(a16 > 0, nb16, Bn)
        pltpu.sync_copy(data_v, acc_sum.at[idx_v.at[0]], add=True)
        pltpu.sync_copy(ones_v, acc_cnt.at[idx_v.at[0]], add=True)
        return carry

    lax.fori_loop(0, my_n, body, 0)
    plsc.subcore_barrier()

    @pl.when(sid == 0)
    def _():
        pltpu.sync_copy(acc_sum, out_sum_hbm.at[cid])
        pltpu.sync_copy(acc_cnt, out_cnt_hbm.at[cid])


def _finish_body(Bn, H, max_steps,
                 psum_ref, pcnt_ref, q_ref, sc_ref, w_ref, g_ref, b_ref,
                 o_ref):
    s = psum_ref[0, :Bn, :] + psum_ref[1, :Bn, :]
    cnt = pcnt_ref[0, :Bn, 0] + pcnt_ref[1, :Bn, 0]
    cnt = jnp.maximum(cnt, 1.0)
    mean = s / cnt[:, None]
    scv = jnp.clip(sc_ref[:, 0], 0, max_steps)
    remaining = max_steps - scv
    oh = (remaining[:, None]
          == lax.broadcasted_iota(jnp.int32, (Bn, max_steps + 1), 1))
    emb = jnp.dot(oh.astype(jnp.float32), w_ref[...],
                  preferred_element_type=jnp.float32)
    st = mean + q_ref[...] + emb
    mu = jnp.mean(st, axis=-1, keepdims=True)
    var = jnp.mean((st - mu) ** 2, axis=-1, keepdims=True)
    o_ref[...] = (st - mu) * lax.rsqrt(var + 1e-5) * g_ref[0, :] + b_ref[0, :]


def kernel(node_tokens, node_batch, active_nodes, question_tokens,
           step_counts, step_emb_weight, ln_gamma, ln_beta):
    N, H = node_tokens.shape
    Bn = question_tokens.shape[0]
    max_steps = step_emb_weight.shape[0] - 1
    num_chunks = N // CH
    assert N % CH == 0

    act = active_nodes.astype(jnp.int32)
    nb = node_batch.astype(jnp.int32)
    zeros_acc = jnp.zeros((Bn + 1, H), jnp.float32)
    ones_rows = jnp.ones((CH, H), jnp.float32)

    mesh = plsc.VectorSubcoreMesh(core_axis_name="c", subcore_axis_name="s")
    seg = functools.partial(
        pl.kernel,
        out_type=[
            jax.ShapeDtypeStruct((NC, Bn + 1, H), jnp.float32),
            jax.ShapeDtypeStruct((NC, Bn + 1, H), jnp.float32),
        ],
        mesh=mesh,
        scratch_types=[
            pltpu.VMEM((CH, H), jnp.float32),
            pltpu.VMEM((CH,), jnp.int32),
            pltpu.VMEM((CH,), jnp.int32),
            pltpu.VMEM((1, CH), jnp.int32),
            pltpu.VMEM((CH, H), jnp.float32),
            pltpu.VMEM_SHARED((Bn + 1, H), jnp.float32),
            pltpu.VMEM_SHARED((Bn + 1, H), jnp.float32),
        ],
    )(functools.partial(_seg_body, num_chunks, Bn, H))

    psum, pcnt = seg(node_tokens, nb, act, zeros_acc, zeros_acc, ones_rows)

    out = pl.pallas_call(
        functools.partial(_finish_body, Bn, H, max_steps),
        out_shape=jax.ShapeDtypeStruct((Bn, H), jnp.float32),
    )(psum, pcnt, question_tokens,
      step_counts.astype(jnp.int32).reshape(Bn, 1), step_emb_weight,
      ln_gamma.reshape(1, H), ln_beta.reshape(1, H))
    return out


# speedup vs baseline: 6.5270x; 1.3612x over previous
import functools

import jax
import jax.numpy as jnp
from jax import lax
from jax.experimental import pallas as pl
from jax.experimental.pallas import tpu as pltpu
from jax.experimental.pallas import tpu_sc as plsc

NC = 2
NS = 16
NW = NC * NS
CH = 128
LANES = 16


def _seg_body(num_chunks, Bn, H,
              nt_hbm, nb_hbm, act_hbm, zsum_hbm, ones_hbm,
              out_sum_hbm, out_cnt_hbm,
              d0, d1, nb_all, act_all, idx_all, ones_v,
              acc_sum, acc_cnt, sem_l0, sem_l1, sem_o):
    cid = lax.axis_index("c")
    sid = lax.axis_index("s")
    wid = sid * NC + cid

    base_n = num_chunks // NW
    rem = num_chunks % NW
    maxn = base_n + (1 if rem else 0)
    s_w = wid * base_n + jnp.minimum(wid, rem)
    my_n = base_n + jnp.where(wid < rem, 1, 0)

    @pl.when(sid == 0)
    def _():
        pltpu.sync_copy(zsum_hbm, acc_sum)
        pltpu.sync_copy(zsum_hbm, acc_cnt)

    pltpu.sync_copy(ones_hbm, ones_v)
    pltpu.sync_copy(nb_hbm.at[pl.ds(s_w * CH, maxn * CH)], nb_all)
    pltpu.sync_copy(act_hbm.at[pl.ds(s_w * CH, maxn * CH)], act_all)

    for j in range(maxn):
        for i in range(CH // LANES):
            sl = pl.ds(j * CH + i * LANES, LANES)
            idx_all[j, pl.ds(i * LANES, LANES)] = jnp.where(
                act_all[sl] > 0, nb_all[sl], Bn)

    plsc.subcore_barrier()

    def load(j, dbuf, sem):
        pltpu.async_copy(nt_hbm.at[pl.ds((s_w + j) * CH, CH)], dbuf, sem)

    def wait_load(dbuf, sem):
        pltpu.make_async_copy(nt_hbm.at[pl.ds(0, CH)], dbuf, sem).wait()

    def drain_ones():
        pltpu.make_async_copy(ones_v, acc_cnt.at[idx_all.at[0]], sem_o).wait()

    load(0, d0, sem_l0)
    load(1, d1, sem_l1)

    def halfstep(j, dbuf, sem):
        wait_load(dbuf, sem)

        @pl.when(j > 0)
        def _():
            drain_ones()

        pltpu.sync_copy(dbuf, acc_sum.at[idx_all.at[j]], add=True)
        pltpu.async_copy(ones_v, acc_cnt.at[idx_all.at[j]], sem_o, add=True)

        @pl.when(j + 2 < my_n)
        def _():
            load(j + 2, dbuf, sem)

    def body(g, carry):
        halfstep(2 * g, d0, sem_l0)
        halfstep(2 * g + 1, d1, sem_l1)
        return carry

    lax.fori_loop(0, my_n // 2, body, 0)

    @pl.when(my_n % 2 == 1)
    def _():
        halfstep(my_n - 1, d0, sem_l0)

    drain_ones()
    plsc.subcore_barrier()

    @pl.when(sid == 0)
    def _():
        pltpu.sync_copy(acc_sum, out_sum_hbm.at[cid])
        pltpu.sync_copy(acc_cnt, out_cnt_hbm.at[cid])


def _finish_body(Bn, H, max_steps,
                 psum_ref, pcnt_ref, q_ref, sc_ref, w_ref, g_ref, b_ref,
                 o_ref):
    s = psum_ref[0, :Bn, :] + psum_ref[1, :Bn, :]
    cnt = pcnt_ref[0, :Bn, 0] + pcnt_ref[1, :Bn, 0]
    cnt = jnp.maximum(cnt, 1.0)
    mean = s / cnt[:, None]
    scv = jnp.clip(sc_ref[:, 0], 0, max_steps)
    remaining = max_steps - scv
    oh = (remaining[:, None]
          == lax.broadcasted_iota(jnp.int32, (Bn, max_steps + 1), 1))
    emb = jnp.dot(oh.astype(jnp.float32), w_ref[...],
                  preferred_element_type=jnp.float32)
    st = mean + q_ref[...] + emb
    mu = jnp.mean(st, axis=-1, keepdims=True)
    var = jnp.mean((st - mu) ** 2, axis=-1, keepdims=True)
    o_ref[...] = (st - mu) * lax.rsqrt(var + 1e-5) * g_ref[0, :] + b_ref[0, :]


def kernel(node_tokens, node_batch, active_nodes, question_tokens,
           step_counts, step_emb_weight, ln_gamma, ln_beta):
    N, H = node_tokens.shape
    Bn = question_tokens.shape[0]
    max_steps = step_emb_weight.shape[0] - 1
    assert N % CH == 0
    num_chunks = N // CH
    maxn = num_chunks // NW + (1 if num_chunks % NW else 0)

    pad = jnp.zeros((CH,), jnp.int32)
    nb = jnp.concatenate([node_batch.astype(jnp.int32), pad])
    act = jnp.concatenate([active_nodes.astype(jnp.int32), pad])
    zeros_acc = jnp.zeros((Bn + 1, H), jnp.float32)
    ones_rows = jnp.ones((CH, H), jnp.float32)

    mesh = plsc.VectorSubcoreMesh(core_axis_name="c", subcore_axis_name="s")
    seg = functools.partial(
        pl.kernel,
        out_type=[
            jax.ShapeDtypeStruct((NC, Bn + 1, H), jnp.float32),
            jax.ShapeDtypeStruct((NC, Bn + 1, H), jnp.float32),
        ],
        mesh=mesh,
        scratch_types=[
            pltpu.VMEM((CH, H), jnp.float32),
            pltpu.VMEM((CH, H), jnp.float32),
            pltpu.VMEM((maxn * CH,), jnp.int32),
            pltpu.VMEM((maxn * CH,), jnp.int32),
            pltpu.VMEM((maxn, CH), jnp.int32),
            pltpu.VMEM((CH, H), jnp.float32),
            pltpu.VMEM_SHARED((Bn + 1, H), jnp.float32),
            pltpu.VMEM_SHARED((Bn + 1, H), jnp.float32),
            pltpu.SemaphoreType.DMA,
            pltpu.SemaphoreType.DMA,
            pltpu.SemaphoreType.DMA,
        ],
    )(functools.partial(_seg_body, num_chunks, Bn, H))

    psum, pcnt = seg(node_tokens, nb, act, zeros_acc, ones_rows)

    out = pl.pallas_call(
        functools.partial(_finish_body, Bn, H, max_steps),
        out_shape=jax.ShapeDtypeStruct((Bn, H), jnp.float32),
    )(psum, pcnt, question_tokens,
      step_counts.astype(jnp.int32).reshape(Bn, 1), step_emb_weight,
      ln_gamma.reshape(1, H), ln_beta.reshape(1, H))
    return out


# speedup vs baseline: 7.3736x; 1.1297x over previous
import functools

import jax
import jax.numpy as jnp
from jax import lax
from jax.experimental import pallas as pl
from jax.experimental.pallas import tpu as pltpu
from jax.experimental.pallas import tpu_sc as plsc

NC = 2
NS = 16
NW = NC * NS
CH = 128
LANES = 16


def _seg_body(num_chunks, Bn, H,
              nt_hbm, nb_hbm, act_hbm, zsum_hbm, ones_hbm,
              out_sum_hbm, out_cnt_hbm,
              d0, d1, d2, nb_all, act_all, idx_all, ones_v,
              acc_sum, acc_cnt, sem_l0, sem_l1, sem_l2,
              sem_d0, sem_d1, sem_d2, sem_o):
    cid = lax.axis_index("c")
    sid = lax.axis_index("s")
    wid = sid * NC + cid

    base_n = num_chunks // NW
    rem = num_chunks % NW
    maxn = base_n + (1 if rem else 0)
    s_w = wid * base_n + jnp.minimum(wid, rem)
    my_n = base_n + jnp.where(wid < rem, 1, 0)

    @pl.when(sid == 0)
    def _():
        pltpu.sync_copy(zsum_hbm, acc_sum)
        pltpu.sync_copy(zsum_hbm, acc_cnt)

    pltpu.sync_copy(ones_hbm, ones_v)
    pltpu.sync_copy(nb_hbm.at[pl.ds(s_w * CH, maxn * CH)], nb_all)
    pltpu.sync_copy(act_hbm.at[pl.ds(s_w * CH, maxn * CH)], act_all)

    for j in range(maxn):
        for i in range(CH // LANES):
            sl = pl.ds(j * CH + i * LANES, LANES)
            idx_all[j, pl.ds(i * LANES, LANES)] = jnp.where(
                act_all[sl] > 0, nb_all[sl], Bn)

    plsc.subcore_barrier()

    bufs = (d0, d1, d2)
    lsems = (sem_l0, sem_l1, sem_l2)
    dsems = (sem_d0, sem_d1, sem_d2)

    def load(j, b):
        pltpu.async_copy(nt_hbm.at[pl.ds((s_w + j) * CH, CH)], bufs[b],
                         lsems[b])

    def wait_load(b):
        pltpu.make_async_copy(nt_hbm.at[pl.ds(0, CH)], bufs[b],
                              lsems[b]).wait()

    def wait_data(b):
        pltpu.make_async_copy(bufs[b], acc_sum.at[idx_all.at[0]],
                              dsems[b]).wait()

    def drain_ones():
        pltpu.make_async_copy(ones_v, acc_cnt.at[idx_all.at[0]], sem_o).wait()

    load(0, 0)
    load(1, 1)

    def step(j, b):
        b2 = (b + 2) % 3
        wait_load(b)

        @pl.when(j > 0)
        def _():
            drain_ones()
            wait_data(b2)

        @pl.when(j + 2 < my_n)
        def _():
            load(j + 2, b2)

        pltpu.async_copy(bufs[b], acc_sum.at[idx_all.at[j]], dsems[b],
                         add=True)
        pltpu.async_copy(ones_v, acc_cnt.at[idx_all.at[j]], sem_o, add=True)

    def body(g, carry):
        step(3 * g, 0)
        step(3 * g + 1, 1)
        step(3 * g + 2, 2)
        return carry

    lax.fori_loop(0, my_n // 3, body, 0)

    cases = sorted({base_n, base_n + (1 if rem else 0)})
    for val in cases:
        for j in range(3 * (val // 3), val):

            @pl.when(my_n == val)
            def _(j=j):
                step(jnp.int32(j), j % 3)

    drain_ones()
    for val in cases:

        @pl.when(my_n == val)
        def _(val=val):
            wait_data((val - 1) % 3)

    plsc.subcore_barrier()

    @pl.when(sid == 0)
    def _():
        pltpu.sync_copy(acc_sum, out_sum_hbm.at[cid])
        pltpu.sync_copy(acc_cnt, out_cnt_hbm.at[cid])


def _finish_body(Bn, H, max_steps,
                 psum_ref, pcnt_ref, q_ref, sc_ref, w_ref, g_ref, b_ref,
                 o_ref):
    s = psum_ref[0, :Bn, :] + psum_ref[1, :Bn, :]
    cnt = pcnt_ref[0, :Bn, 0] + pcnt_ref[1, :Bn, 0]
    cnt = jnp.maximum(cnt, 1.0)
    mean = s / cnt[:, None]
    scv = jnp.clip(sc_ref[:, 0], 0, max_steps)
    remaining = max_steps - scv
    oh = (remaining[:, None]
          == lax.broadcasted_iota(jnp.int32, (Bn, max_steps + 1), 1))
    emb = jnp.dot(oh.astype(jnp.float32), w_ref[...],
                  preferred_element_type=jnp.float32)
    st = mean + q_ref[...] + emb
    mu = jnp.mean(st, axis=-1, keepdims=True)
    var = jnp.mean((st - mu) ** 2, axis=-1, keepdims=True)
    o_ref[...] = (st - mu) * lax.rsqrt(var + 1e-5) * g_ref[0, :] + b_ref[0, :]


def kernel(node_tokens, node_batch, active_nodes, question_tokens,
           step_counts, step_emb_weight, ln_gamma, ln_beta):
    N, H = node_tokens.shape
    Bn = question_tokens.shape[0]
    max_steps = step_emb_weight.shape[0] - 1
    assert N % CH == 0
    num_chunks = N // CH
    maxn = num_chunks // NW + (1 if num_chunks % NW else 0)

    pad = jnp.zeros((CH,), jnp.int32)
    nb = jnp.concatenate([node_batch.astype(jnp.int32), pad])
    act = jnp.concatenate([active_nodes.astype(jnp.int32), pad])
    zeros_acc = jnp.zeros((Bn + 1, H), jnp.float32)
    ones_rows = jnp.ones((CH, H), jnp.float32)

    mesh = plsc.VectorSubcoreMesh(core_axis_name="c", subcore_axis_name="s")
    seg = functools.partial(
        pl.kernel,
        out_type=[
            jax.ShapeDtypeStruct((NC, Bn + 1, H), jnp.float32),
            jax.ShapeDtypeStruct((NC, Bn + 1, H), jnp.float32),
        ],
        mesh=mesh,
        scratch_types=[
            pltpu.VMEM((CH, H), jnp.float32),
            pltpu.VMEM((CH, H), jnp.float32),
            pltpu.VMEM((CH, H), jnp.float32),
            pltpu.VMEM((maxn * CH,), jnp.int32),
            pltpu.VMEM((maxn * CH,), jnp.int32),
            pltpu.VMEM((maxn, CH), jnp.int32),
            pltpu.VMEM((CH, H), jnp.float32),
            pltpu.VMEM_SHARED((Bn + 1, H), jnp.float32),
            pltpu.VMEM_SHARED((Bn + 1, H), jnp.float32),
            pltpu.SemaphoreType.DMA,
            pltpu.SemaphoreType.DMA,
            pltpu.SemaphoreType.DMA,
            pltpu.SemaphoreType.DMA,
            pltpu.SemaphoreType.DMA,
            pltpu.SemaphoreType.DMA,
            pltpu.SemaphoreType.DMA,
        ],
    )(functools.partial(_seg_body, num_chunks, Bn, H))

    psum, pcnt = seg(node_tokens, nb, act, zeros_acc, ones_rows)

    out = pl.pallas_call(
        functools.partial(_finish_body, Bn, H, max_steps),
        out_shape=jax.ShapeDtypeStruct((Bn, H), jnp.float32),
    )(psum, pcnt, question_tokens,
      step_counts.astype(jnp.int32).reshape(Bn, 1), step_emb_weight,
      ln_gamma.reshape(1, H), ln_beta.reshape(1, H))
    return out


# speedup vs baseline: 7.3796x; 1.0008x over previous
import functools

import jax
import jax.numpy as jnp
from jax import lax
from jax.experimental import pallas as pl
from jax.experimental.pallas import tpu as pltpu
from jax.experimental.pallas import tpu_sc as plsc

NC = 2
NS = 16
NW = NC * NS
CH = 128
LANES = 16


def _seg_body(num_chunks, Bn, H,
              nt_hbm, nb_hbm, act_hbm, zsum_hbm, ones_hbm,
              out_sum_hbm, out_cnt_hbm,
              d0, d1, d2, nb_all, act_all, idx_all, ones_v,
              acc_sum, acc_cnt, sem_l0, sem_l1, sem_l2,
              sem_d0, sem_d1, sem_d2, sem_o0, sem_o1, sem_o2):
    cid = lax.axis_index("c")
    sid = lax.axis_index("s")
    wid = sid * NC + cid

    base_n = num_chunks // NW
    rem = num_chunks % NW
    maxn = base_n + (1 if rem else 0)
    s_w = wid * base_n + jnp.minimum(wid, rem)
    my_n = base_n + jnp.where(wid < rem, 1, 0)

    @pl.when(sid == 0)
    def _():
        pltpu.sync_copy(zsum_hbm, acc_sum)
        pltpu.sync_copy(zsum_hbm, acc_cnt)

    pltpu.sync_copy(ones_hbm, ones_v)
    pltpu.sync_copy(nb_hbm.at[pl.ds(s_w * CH, maxn * CH)], nb_all)
    pltpu.sync_copy(act_hbm.at[pl.ds(s_w * CH, maxn * CH)], act_all)

    for j in range(maxn):
        for i in range(CH // LANES):
            sl = pl.ds(j * CH + i * LANES, LANES)
            idx_all[j, pl.ds(i * LANES, LANES)] = jnp.where(
                act_all[sl] > 0, nb_all[sl], Bn)

    plsc.subcore_barrier()

    bufs = (d0, d1, d2)
    lsems = (sem_l0, sem_l1, sem_l2)
    dsems = (sem_d0, sem_d1, sem_d2)
    osems = (sem_o0, sem_o1, sem_o2)

    def load(j, b):
        pltpu.async_copy(nt_hbm.at[pl.ds((s_w + j) * CH, CH)], bufs[b],
                         lsems[b])

    def wait_load(b):
        pltpu.make_async_copy(nt_hbm.at[pl.ds(0, CH)], bufs[b],
                              lsems[b]).wait()

    def wait_data(b):
        pltpu.make_async_copy(bufs[b], acc_sum.at[idx_all.at[0]],
                              dsems[b]).wait()

    def drain_ones(b):
        pltpu.make_async_copy(ones_v, acc_cnt.at[idx_all.at[0]],
                              osems[b]).wait()

    load(0, 0)
    load(1, 1)

    def step(j, b):
        b2 = (b + 2) % 3
        wait_load(b)

        @pl.when(j > 0)
        def _():
            wait_data(b2)

        @pl.when(j >= 3)
        def _():
            drain_ones(b)

        @pl.when(j + 2 < my_n)
        def _():
            load(j + 2, b2)

        pltpu.async_copy(bufs[b], acc_sum.at[idx_all.at[j]], dsems[b],
                         add=True)
        pltpu.async_copy(ones_v, acc_cnt.at[idx_all.at[j]], osems[b],
                        add=True)

    def body(g, carry):
        step(3 * g, 0)
        step(3 * g + 1, 1)
        step(3 * g + 2, 2)
        return carry

    lax.fori_loop(0, my_n // 3, body, 0)

    cases = sorted({base_n, base_n + (1 if rem else 0)})
    for val in cases:
        for j in range(3 * (val // 3), val):

            @pl.when(my_n == val)
            def _(j=j):
                step(jnp.int32(j), j % 3)

    drain_ones(0)
    drain_ones(1)
    drain_ones(2)
    for val in cases:

        @pl.when(my_n == val)
        def _(val=val):
            wait_data((val - 1) % 3)

    plsc.subcore_barrier()

    @pl.when(sid == 0)
    def _():
        pltpu.sync_copy(acc_sum, out_sum_hbm.at[cid])
        pltpu.sync_copy(acc_cnt, out_cnt_hbm.at[cid])


def _finish_body(Bn, H, max_steps,
                 psum_ref, pcnt_ref, q_ref, sc_ref, w_ref, g_ref, b_ref,
                 o_ref):
    s = psum_ref[0, :Bn, :] + psum_ref[1, :Bn, :]
    cnt = pcnt_ref[0, :Bn, 0] + pcnt_ref[1, :Bn, 0]
    cnt = jnp.maximum(cnt, 1.0)
    mean = s / cnt[:, None]
    scv = jnp.clip(sc_ref[:, 0], 0, max_steps)
    remaining = max_steps - scv
    oh = (remaining[:, None]
          == lax.broadcasted_iota(jnp.int32, (Bn, max_steps + 1), 1))
    emb = jnp.dot(oh.astype(jnp.float32), w_ref[...],
                  preferred_element_type=jnp.float32)
    st = mean + q_ref[...] + emb
    mu = jnp.mean(st, axis=-1, keepdims=True)
    var = jnp.mean((st - mu) ** 2, axis=-1, keepdims=True)
    o_ref[...] = (st - mu) * lax.rsqrt(var + 1e-5) * g_ref[0, :] + b_ref[0, :]


def kernel(node_tokens, node_batch, active_nodes, question_tokens,
           step_counts, step_emb_weight, ln_gamma, ln_beta):
    N, H = node_tokens.shape
    Bn = question_tokens.shape[0]
    max_steps = step_emb_weight.shape[0] - 1
    assert N % CH == 0
    num_chunks = N // CH
    maxn = num_chunks // NW + (1 if num_chunks % NW else 0)

    pad = jnp.zeros((CH,), jnp.int32)
    nb = jnp.concatenate([node_batch.astype(jnp.int32), pad])
    act = jnp.concatenate([active_nodes.astype(jnp.int32), pad])
    zeros_acc = jnp.zeros((Bn + 1, H), jnp.float32)
    ones_rows = jnp.ones((CH, H), jnp.float32)

    mesh = plsc.VectorSubcoreMesh(core_axis_name="c", subcore_axis_name="s")
    seg = functools.partial(
        pl.kernel,
        out_type=[
            jax.ShapeDtypeStruct((NC, Bn + 1, H), jnp.float32),
            jax.ShapeDtypeStruct((NC, Bn + 1, H), jnp.float32),
        ],
        mesh=mesh,
        scratch_types=[
            pltpu.VMEM((CH, H), jnp.float32),
            pltpu.VMEM((CH, H), jnp.float32),
            pltpu.VMEM((CH, H), jnp.float32),
            pltpu.VMEM((maxn * CH,), jnp.int32),
            pltpu.VMEM((maxn * CH,), jnp.int32),
            pltpu.VMEM((maxn, CH), jnp.int32),
            pltpu.VMEM((CH, H), jnp.float32),
            pltpu.VMEM_SHARED((Bn + 1, H), jnp.float32),
            pltpu.VMEM_SHARED((Bn + 1, H), jnp.float32),
            pltpu.SemaphoreType.DMA,
            pltpu.SemaphoreType.DMA,
            pltpu.SemaphoreType.DMA,
            pltpu.SemaphoreType.DMA,
            pltpu.SemaphoreType.DMA,
            pltpu.SemaphoreType.DMA,
            pltpu.SemaphoreType.DMA,
            pltpu.SemaphoreType.DMA,
            pltpu.SemaphoreType.DMA,
        ],
    )(functools.partial(_seg_body, num_chunks, Bn, H))

    psum, pcnt = seg(node_tokens, nb, act, zeros_acc, ones_rows)

    out = pl.pallas_call(
        functools.partial(_finish_body, Bn, H, max_steps),
        out_shape=jax.ShapeDtypeStruct((Bn, H), jnp.float32),
    )(psum, pcnt, question_tokens,
      step_counts.astype(jnp.int32).reshape(Bn, 1), step_emb_weight,
      ln_gamma.reshape(1, H), ln_beta.reshape(1, H))
    return out
